# baseline (speedup 1.0000x reference)
import functools

import jax
import jax.numpy as jnp
from jax import lax
from jax.experimental import pallas as pl
from jax.experimental.pallas import tpu as pltpu
from jax.experimental.pallas import tpu_sc as plsc

_BR = 256
_K = 5
_NT = 2048
_NSC = 1024
_NC = 2
_NS = 16
_L = 16
_NW = _NC * _NS
_RPW = _NSC // _NW
_CH = _NT // _L


def _tc_body(q_ref, l_ref, t_ref, c_ref, o_ref):
    nt = t_ref.shape[1]
    qx = q_ref[:, 0:1]
    qy = q_ref[:, 1:2]
    tx = t_ref[0:1, :]
    ty = t_ref[1:2, :]
    dx = qx - tx
    dy = qy - ty
    s2 = dx * dx + dy * dy
    d = jnp.sqrt(s2)

    iota = lax.broadcasted_iota(jnp.int32, s2.shape, 1)
    cur = d
    total = jnp.zeros((s2.shape[0], 1), jnp.float32)
    for i in range(_K):
        m = jnp.min(cur, axis=1, keepdims=True)
        total = total + m
        if i < _K - 1:
            pos = jnp.min(jnp.where(cur <= m, iota, nt), axis=1,
                          keepdims=True)
            cur = jnp.where(iota == pos, jnp.float32(jnp.inf), cur)
    delta = total / jnp.float32(_K)

    p0 = jax.nn.sigmoid(l_ref[:, 0:1] - l_ref[:, 1:2])
    tcls = c_ref[0:1, :]
    cls_cost = p0 + tcls * (1.0 - 2.0 * p0)

    w = jnp.exp(s2 * (-1.0 / 50.0))
    cost_point = jnp.where(d < delta, d * w, d)
    o_ref[:, :] = cost_point - cls_cost


def _nr_sqrt(s2):
    i = lax.bitcast_convert_type(s2, jnp.int32)
    i = jnp.int32(0x5F3759DF) - lax.shift_right_arithmetic(i, 1)
    y = lax.bitcast_convert_type(i, jnp.float32)
    for _ in range(3):
        y = y * (1.5 - 0.5 * s2 * y * y)
    return jnp.where(s2 > 0.0, s2 * y, 0.0)


_NSTREAM = 4


def _sc_body(qx_hbm, qy_hbm, l0_hbm, l1_hbm, tgt_hbm, out_hbm,
             qxv, qyv, l0v, l1v, tgtv, rowsv):
    wid = lax.axis_index("s") * _NC + lax.axis_index("c")
    base = wid * _RPW
    pltpu.sync_copy(qx_hbm.at[pl.ds(base, _RPW)], qxv)
    pltpu.sync_copy(qy_hbm.at[pl.ds(base, _RPW)], qyv)
    pltpu.sync_copy(l0_hbm.at[pl.ds(base, _RPW)], l0v)
    pltpu.sync_copy(l1_hbm.at[pl.ds(base, _RPW)], l1v)
    pltpu.sync_copy(tgt_hbm, tgtv)

    inf = jnp.full((_L,), jnp.inf, jnp.float32)

    def group_body(g, carry):
        goff = pl.multiple_of(g * _L, _L)
        qxg = qxv[pl.ds(goff, _L)]
        qyg = qyv[pl.ds(goff, _L)]
        l0g = l0v[pl.ds(goff, _L)]
        l1g = l1v[pl.ds(goff, _L)]

        def ch1(c, streams):
            off = pl.multiple_of(c * _L, _L)
            tx = tgtv[0, pl.ds(off, _L)]
            ty = tgtv[1, pl.ds(off, _L)]
            s2s = []
            for j in range(_L):
                dx = qxg - tx[j]
                dy = qyg - ty[j]
                s2s.append(dx * dx + dy * dy)
            nxt_streams = []
            for st in range(_NSTREAM):
                ms = streams[st * _K:(st + 1) * _K]
                for j in range(st, _L, _NSTREAM):
                    new = s2s[j]
                    nxt = []
                    for mk in ms:
                        lo = jnp.minimum(mk, new)
                        new = jnp.maximum(mk, new)
                        nxt.append(lo)
                    ms = nxt
                nxt_streams.extend(ms)
            return tuple(nxt_streams)

        streams = lax.fori_loop(0, _CH, ch1, (inf,) * (_K * _NSTREAM))

        ms = list(streams[:_K])
        for st in range(1, _NSTREAM):
            for new in streams[st * _K:(st + 1) * _K]:
                nxt = []
                for mk in ms:
                    lo = jnp.minimum(mk, new)
                    new = jnp.maximum(mk, new)
                    nxt.append(lo)
                ms = nxt

        total = jnp.zeros((_L,), jnp.float32)
        for mk in ms:
            total = total + _nr_sqrt(mk)
        delta_g = total / jnp.float32(_K)
        p0_g = 1.0 / (1.0 + jnp.exp(l1g - l0g))
        u_g = 1.0 - 2.0 * p0_g

        qxs = [qxg[j] for j in range(_L)]
        qys = [qyg[j] for j in range(_L)]
        djs = [delta_g[j] for j in range(_L)]
        p0s = [p0_g[j] for j in range(_L)]
        us = [u_g[j] for j in range(_L)]

        def ch2(c, carry2):
            off = pl.multiple_of(c * _L, _L)
            tx = tgtv[0, pl.ds(off, _L)]
            ty = tgtv[1, pl.ds(off, _L)]
            tc = tgtv[2, pl.ds(off, _L)]
            for j in range(_L):
                dx = tx - qxs[j]
                dy = ty - qys[j]
                s2 = dx * dx + dy * dy
                d = _nr_sqrt(s2)
                w = jnp.exp(s2 * (-1.0 / 50.0))
                cp = jnp.where(d < djs[j], d * w, d)
                rowsv[j, pl.ds(off, _L)] = cp - (p0s[j] + tc * us[j])
            return carry2

        lax.fori_loop(0, _CH, ch2, 0)

        pltpu.sync_copy(rowsv, out_hbm.at[pl.ds(base + goff, _L)])
        return carry

    lax.fori_loop(0, _RPW // _L, group_body, 0)


@jax.jit
def kernel(pred_logits, pred_points, tgt_points, tgt_ids):
    bs, nq, _ = pred_logits.shape
    nt = tgt_points.shape[0]
    nq_flat = bs * nq
    q = pred_points.reshape(nq_flat, 2)
    logits = pred_logits.reshape(nq_flat, 2)
    t_t = tgt_points.T
    clsf = tgt_ids.astype(jnp.float32)
    cls = clsf.reshape(1, nt)

    tgt3 = jnp.concatenate([t_t, cls], axis=0)
    mesh = plsc.VectorSubcoreMesh(core_axis_name="c", subcore_axis_name="s",
                                  num_cores=_NC, num_subcores=_NS)
    sc_out = pl.kernel(
        _sc_body,
        out_type=jax.ShapeDtypeStruct((_NSC, nt), jnp.float32),
        mesh=mesh,
        scratch_types=[
            pltpu.VMEM((_RPW,), jnp.float32),
            pltpu.VMEM((_RPW,), jnp.float32),
            pltpu.VMEM((_RPW,), jnp.float32),
            pltpu.VMEM((_RPW,), jnp.float32),
            pltpu.VMEM((3, nt), jnp.float32),
            pltpu.VMEM((_L, nt), jnp.float32),
        ],
    )(q[:_NSC, 0], q[:_NSC, 1], logits[:_NSC, 0], logits[:_NSC, 1], tgt3)

    n_tc_blocks = (nq_flat - _NSC) // _BR
    off_blocks = _NSC // _BR
    tc_out = pl.pallas_call(
        _tc_body,
        grid=(n_tc_blocks,),
        in_specs=[
            pl.BlockSpec((_BR, 2), lambda i: (i + off_blocks, 0)),
            pl.BlockSpec((_BR, 2), lambda i: (i + off_blocks, 0)),
            pl.BlockSpec((2, nt), lambda i: (0, 0)),
            pl.BlockSpec((1, nt), lambda i: (0, 0)),
        ],
        out_specs=pl.BlockSpec((_BR, nt), lambda i: (i + off_blocks, 0)),
        out_shape=jax.ShapeDtypeStruct((nq_flat, nt), jnp.float32),
    )(q, logits, t_t, cls)

    def _merge_body(_, sc_ref, o_ref):
        o_ref[:, :] = sc_ref[:, :]

    out = pl.pallas_call(
        _merge_body,
        grid=(_NSC // _BR,),
        in_specs=[
            pl.BlockSpec(memory_space=pl.ANY),
            pl.BlockSpec((_BR, nt), lambda i: (i, 0)),
        ],
        out_specs=pl.BlockSpec((_BR, nt), lambda i: (i, 0)),
        out_shape=jax.ShapeDtypeStruct((nq_flat, nt), jnp.float32),
        input_output_aliases={0: 0},
    )(tc_out, sc_out)
    return out.reshape(bs, nq, nt)


# speedup vs baseline: 1.2846x; 1.2846x over previous
import jax
import jax.numpy as jnp
from jax import lax
from jax.experimental import pallas as pl

_BR = 256
_K = 5
_W = 128


def _cost_body(q_ref, l_ref, t_ref, c_ref, o_ref):
    nt = t_ref.shape[1]
    br = q_ref.shape[0]
    qx = q_ref[:, 0:1]
    qy = q_ref[:, 1:2]
    tx = t_ref[0:1, :]
    ty = t_ref[1:2, :]
    dx = qx - tx
    dy = qy - ty
    s2 = dx * dx + dy * dy
    d = jnp.sqrt(s2)

    ms = [jnp.full((br, _W), jnp.inf, jnp.float32) for _ in range(_K)]
    for c in range(nt // _W):
        new = d[:, c * _W:(c + 1) * _W]
        for k in range(_K):
            lo = jnp.minimum(ms[k], new)
            new = jnp.maximum(ms[k], new)
            ms[k] = lo

    cand = jnp.concatenate(ms, axis=1)
    ncand = _K * _W
    iota = lax.broadcasted_iota(jnp.int32, (br, ncand), 1)
    total = jnp.zeros((br, 1), jnp.float32)
    for i in range(_K):
        m = jnp.min(cand, axis=1, keepdims=True)
        total = total + m
        if i < _K - 1:
            pos = jnp.min(jnp.where(cand <= m, iota, ncand), axis=1,
                          keepdims=True)
            cand = jnp.where(iota == pos, jnp.float32(jnp.inf), cand)
    delta = total / jnp.float32(_K)

    p0 = jax.nn.sigmoid(l_ref[:, 0:1] - l_ref[:, 1:2])
    tcls = c_ref[0:1, :]
    cls_cost = p0 + tcls * (1.0 - 2.0 * p0)

    w = jnp.exp(s2 * (-1.0 / 50.0))
    cost_point = jnp.where(d < delta, d * w, d)
    o_ref[:, :] = cost_point - cls_cost


@jax.jit
def kernel(pred_logits, pred_points, tgt_points, tgt_ids):
    bs, nq, _ = pred_logits.shape
    nt = tgt_points.shape[0]
    nq_flat = bs * nq
    q = pred_points.reshape(nq_flat, 2)
    logits = pred_logits.reshape(nq_flat, 2)
    t_t = tgt_points.T
    cls = tgt_ids.astype(jnp.float32).reshape(1, nt)

    out = pl.pallas_call(
        _cost_body,
        grid=(nq_flat // _BR,),
        in_specs=[
            pl.BlockSpec((_BR, 2), lambda i: (i, 0)),
            pl.BlockSpec((_BR, 2), lambda i: (i, 0)),
            pl.BlockSpec((2, nt), lambda i: (0, 0)),
            pl.BlockSpec((1, nt), lambda i: (0, 0)),
        ],
        out_specs=pl.BlockSpec((_BR, nt), lambda i: (i, 0)),
        out_shape=jax.ShapeDtypeStruct((nq_flat, nt), jnp.float32),
    )(q, logits, t_t, cls)
    return out.reshape(bs, nq, nt)


# speedup vs baseline: 1.4092x; 1.0970x over previous
import jax
import jax.numpy as jnp
from jax import lax
from jax.experimental import pallas as pl

_BR = 256
_K = 5
_W = 128


def _cost_body(q_ref, l_ref, t_ref, c_ref, o_ref):
    nt = t_ref.shape[1]
    br = q_ref.shape[0]
    qx = q_ref[:, 0:1]
    qy = q_ref[:, 1:2]
    tx = t_ref[0:1, :]
    ty = t_ref[1:2, :]
    dx = qx - tx
    dy = qy - ty
    s2 = dx * dx + dy * dy
    d = jnp.sqrt(s2)

    ms = [jnp.full((br, _W), jnp.inf, jnp.float32) for _ in range(_K)]
    for c in range(nt // _W):
        new = d[:, c * _W:(c + 1) * _W]
        for k in range(_K):
            lo = jnp.minimum(ms[k], new)
            new = jnp.maximum(ms[k], new)
            ms[k] = lo

    iota = lax.broadcasted_iota(jnp.int32, (br, _W), 1)
    inf = jnp.float32(jnp.inf)
    total = jnp.zeros((br, 1), jnp.float32)
    for i in range(_K):
        m = jnp.min(ms[0], axis=1, keepdims=True)
        total = total + m
        if i < _K - 1:
            pos = jnp.min(jnp.where(ms[0] <= m, iota, _W), axis=1,
                          keepdims=True)
            sel = iota == pos
            for k in range(_K - 1):
                ms[k] = jnp.where(sel, ms[k + 1], ms[k])
            ms[_K - 1] = jnp.where(sel, inf, ms[_K - 1])
    delta = total / jnp.float32(_K)

    p0 = jax.nn.sigmoid(l_ref[:, 0:1] - l_ref[:, 1:2])
    tcls = c_ref[0:1, :]
    cls_cost = p0 + tcls * (1.0 - 2.0 * p0)

    w = jnp.exp(s2 * (-1.0 / 50.0))
    cost_point = jnp.where(d < delta, d * w, d)
    o_ref[:, :] = cost_point - cls_cost


@jax.jit
def kernel(pred_logits, pred_points, tgt_points, tgt_ids):
    bs, nq, _ = pred_logits.shape
    nt = tgt_points.shape[0]
    nq_flat = bs * nq
    q = pred_points.reshape(nq_flat, 2)
    logits = pred_logits.reshape(nq_flat, 2)
    t_t = tgt_points.T
    cls = tgt_ids.astype(jnp.float32).reshape(1, nt)

    out = pl.pallas_call(
        _cost_body,
        grid=(nq_flat // _BR,),
        in_specs=[
            pl.BlockSpec((_BR, 2), lambda i: (i, 0)),
            pl.BlockSpec((_BR, 2), lambda i: (i, 0)),
            pl.BlockSpec((2, nt), lambda i: (0, 0)),
            pl.BlockSpec((1, nt), lambda i: (0, 0)),
        ],
        out_specs=pl.BlockSpec((_BR, nt), lambda i: (i, 0)),
        out_shape=jax.ShapeDtypeStruct((nq_flat, nt), jnp.float32),
    )(q, logits, t_t, cls)
    return out.reshape(bs, nq, nt)


# speedup vs baseline: 1.4759x; 1.0473x over previous
import jax
import jax.numpy as jnp
from jax import lax
from jax.experimental import pallas as pl

_BR = 512
_K = 5
_W = 128


def _cost_body(q_ref, l_ref, t_ref, c_ref, o_ref):
    nt = t_ref.shape[1]
    br = q_ref.shape[0]
    qx = q_ref[:, 0:1]
    qy = q_ref[:, 1:2]
    tx = t_ref[0:1, :]
    ty = t_ref[1:2, :]
    dx = qx - tx
    dy = qy - ty
    s2 = dx * dx + dy * dy
    d = jnp.sqrt(s2)

    ms = [jnp.full((br, _W), jnp.inf, jnp.float32) for _ in range(_K)]
    for c in range(nt // _W):
        new = d[:, c * _W:(c + 1) * _W]
        for k in range(_K):
            lo = jnp.minimum(ms[k], new)
            new = jnp.maximum(ms[k], new)
            ms[k] = lo

    iota = lax.broadcasted_iota(jnp.int32, (br, _W), 1)
    inf = jnp.float32(jnp.inf)
    total = jnp.zeros((br, 1), jnp.float32)
    for i in range(_K):
        m = jnp.min(ms[0], axis=1, keepdims=True)
        total = total + m
        if i < _K - 1:
            pos = jnp.min(jnp.where(ms[0] <= m, iota, _W), axis=1,
                          keepdims=True)
            sel = iota == pos
            for k in range(_K - 1):
                ms[k] = jnp.where(sel, ms[k + 1], ms[k])
            ms[_K - 1] = jnp.where(sel, inf, ms[_K - 1])
    delta = total / jnp.float32(_K)

    p0 = jax.nn.sigmoid(l_ref[:, 0:1] - l_ref[:, 1:2])
    tcls = c_ref[0:1, :]
    cls_cost = p0 + tcls * (1.0 - 2.0 * p0)

    w = jnp.exp(s2 * (-1.0 / 50.0))
    cost_point = jnp.where(d < delta, d * w, d)
    o_ref[:, :] = cost_point - cls_cost


@jax.jit
def kernel(pred_logits, pred_points, tgt_points, tgt_ids):
    bs, nq, _ = pred_logits.shape
    nt = tgt_points.shape[0]
    nq_flat = bs * nq
    q = pred_points.reshape(nq_flat, 2)
    logits = pred_logits.reshape(nq_flat, 2)
    t_t = tgt_points.T
    cls = tgt_ids.astype(jnp.float32).reshape(1, nt)

    out = pl.pallas_call(
        _cost_body,
        grid=(nq_flat // _BR,),
        in_specs=[
            pl.BlockSpec((_BR, 2), lambda i: (i, 0)),
            pl.BlockSpec((_BR, 2), lambda i: (i, 0)),
            pl.BlockSpec((2, nt), lambda i: (0, 0)),
            pl.BlockSpec((1, nt), lambda i: (0, 0)),
        ],
        out_specs=pl.BlockSpec((_BR, nt), lambda i: (i, 0)),
        out_shape=jax.ShapeDtypeStruct((nq_flat, nt), jnp.float32),
    )(q, logits, t_t, cls)
    return out.reshape(bs, nq, nt)
